# speedup vs baseline: 2.5374x; 2.5374x over previous
import jax
import jax.numpy as jnp
from jax import lax
from jax.experimental import pallas as pl
from jax.experimental.pallas import tpu as pltpu

B, T, N = 16, 512, 512
FS = 32
NS = N // FS
SP = NS + 1
PAD = -1.0


def _body(psl_ref, spikes_ref, tm_ref, sm_ref,
          patches_ref, smask_ref, tmask_ref, ss_ref, ts_ref):
    b = pl.program_id(0)
    p = psl_ref[b]
    psl = p % FS
    sidx = p // FS
    do_fix = (psl != 0) & (sidx < NS)

    x = spikes_ref[0]
    lane = lax.broadcasted_iota(jnp.int32, (T, N), 1)
    g = lane // FS
    j = lane - g * FS
    prev = jnp.concatenate(
        [jnp.full((T, FS), PAD, jnp.float32), x[:, : N - FS]], axis=1)
    blended = jnp.where(j < psl, x, prev)
    fixmask = (g == sidx) & do_fix
    patches_ref[0] = jnp.where(fixmask, blended, x)

    li = lax.broadcasted_iota(jnp.int32, (T, SP), 1)

    tm = tm_ref[0]
    tmask_ref[0] = jnp.where(li == 0, 1, jnp.broadcast_to(tm, (T, SP)))

    sm = sm_ref[0]
    s_any = jnp.max(sm, axis=0, keepdims=True)
    s_row = jnp.concatenate(
        [jnp.ones((1, 1), jnp.int32), s_any], axis=1)
    smask_ref[0] = jnp.broadcast_to(s_row, (T, SP))

    ss_ref[0] = li
    ts_ref[0] = lax.broadcasted_iota(jnp.int32, (T, SP), 0)


def kernel(spikes, pad_space_len, pad_time_len, time_attn_mask,
           space_attn_mask):
    del pad_time_len
    tm3 = time_attn_mask.reshape(B, T, 1)
    sm3 = space_attn_mask.reshape(B, NS, FS).transpose(0, 2, 1)

    grid_spec = pltpu.PrefetchScalarGridSpec(
        num_scalar_prefetch=1,
        grid=(B,),
        in_specs=[
            pl.BlockSpec((1, T, N), lambda b, psl: (b, 0, 0)),
            pl.BlockSpec((1, T, 1), lambda b, psl: (b, 0, 0)),
            pl.BlockSpec((1, FS, NS), lambda b, psl: (b, 0, 0)),
        ],
        out_specs=[
            pl.BlockSpec((1, T, N), lambda b, psl: (b, 0, 0)),
            pl.BlockSpec((1, T, SP), lambda b, psl: (b, 0, 0)),
            pl.BlockSpec((1, T, SP), lambda b, psl: (b, 0, 0)),
            pl.BlockSpec((1, T, SP), lambda b, psl: (b, 0, 0)),
            pl.BlockSpec((1, T, SP), lambda b, psl: (b, 0, 0)),
        ],
    )
    patches, smask, tmask, ss, ts = pl.pallas_call(
        _body,
        grid_spec=grid_spec,
        out_shape=[
            jax.ShapeDtypeStruct((B, T, N), jnp.float32),
            jax.ShapeDtypeStruct((B, T, SP), jnp.int32),
            jax.ShapeDtypeStruct((B, T, SP), jnp.int32),
            jax.ShapeDtypeStruct((B, T, SP), jnp.int32),
            jax.ShapeDtypeStruct((B, T, SP), jnp.int32),
        ],
        compiler_params=pltpu.CompilerParams(
            dimension_semantics=("arbitrary",),
        ),
    )(pad_space_len, spikes, tm3, sm3)

    return (patches.reshape(B, T * NS, FS),
            smask.reshape(B, T * SP),
            tmask.reshape(B, T * SP),
            ss.reshape(B, T * SP),
            ts.reshape(B, T * SP))
